# baseline (speedup 1.0000x reference)
import functools

import jax
import jax.numpy as jnp
from jax.experimental import pallas as pl
from jax.experimental.pallas import tpu as pltpu

_GSHARD_W = 0.0
_IMPORTANCE_W = 1.0


def _tile_body(x_ref, w_ref, gates_ref, aux_ref, gshard_ref, imp_ref,
               oh_sum, g_sum,
               *, num_batch, num_experts, num_tiles, seq_len, tile_f):
    i = pl.program_id(0)
    tiles_per_batch = seq_len // tile_f
    w = w_ref[...]
    x = x_ref[...]
    logits = jax.lax.dot_general(
        x, w, (((1,), (0,)), ((), ())),
        preferred_element_type=jnp.float32)
    m = jnp.max(logits, axis=1, keepdims=True)
    e = jnp.exp(logits - m)
    s = jnp.sum(e, axis=1, keepdims=True)
    gates = e / s
    gates_ref[...] = gates
    lane = jax.lax.broadcasted_iota(jnp.int32, logits.shape, 1)
    eq = logits == m
    amin = jnp.min(jnp.where(eq, lane, num_experts), axis=1, keepdims=True)
    onehot = (lane == amin).astype(jnp.float32)

    k = i % tiles_per_batch
    off = k * tile_f

    @pl.when(i < tiles_per_batch)
    def _init():
        oh_sum[pl.ds(off, tile_f), :] = onehot
        g_sum[pl.ds(off, tile_f), :] = gates

    @pl.when(i >= tiles_per_batch)
    def _accum():
        oh_sum[pl.ds(off, tile_f), :] += onehot
        g_sum[pl.ds(off, tile_f), :] += gates

    @pl.when(i == num_tiles - 1)
    def _finalize():
        gs = g_sum[...]
        imp = jnp.sum(gs, axis=0, keepdims=True)
        mean = jnp.sum(imp) / num_experts
        var = jnp.sum((imp - mean) ** 2) / (num_experts - 1)
        imp_loss = var / (mean * mean)
        gsh = jnp.sum(oh_sum[...] * gs)
        gshard = gsh * (num_experts / (seq_len * num_batch * num_batch))
        total_w = _GSHARD_W + _IMPORTANCE_W
        aux_loss = (_GSHARD_W * gshard + _IMPORTANCE_W * imp_loss) / total_w
        imp_ref[...] = jnp.reshape(imp_loss, (1, 1))
        gshard_ref[...] = jnp.reshape(gshard, (1, 1))
        aux_ref[...] = jnp.reshape(aux_loss, (1, 1))


@functools.partial(jax.jit, static_argnames=("tile_f",))
def _router(x, W, tile_f=1024):
    B, S, H = x.shape
    E = W.shape[0]
    R = B * S
    num_tiles = R // tile_f
    xf = x.reshape(R, H)

    tile_body = functools.partial(
        _tile_body, num_batch=B, num_experts=E, num_tiles=num_tiles,
        seq_len=S, tile_f=tile_f)
    scalar_shape = jax.ShapeDtypeStruct((1, 1), jnp.float32)
    scalar_spec = pl.BlockSpec((1, 1), lambda i: (0, 0))
    gates, aux, gshard, imp = pl.pallas_call(
        tile_body,
        grid=(num_tiles,),
        in_specs=[
            pl.BlockSpec((tile_f, H), lambda i: (i, 0)),
            pl.BlockSpec((H, E), lambda i: (0, 0)),
        ],
        out_specs=(
            pl.BlockSpec((tile_f, E), lambda i: (i, 0)),
            scalar_spec, scalar_spec, scalar_spec,
        ),
        out_shape=(
            jax.ShapeDtypeStruct((R, E), jnp.float32),
            scalar_shape, scalar_shape, scalar_shape,
        ),
        scratch_shapes=[
            pltpu.VMEM((S, E), jnp.float32),
            pltpu.VMEM((S, E), jnp.float32),
        ],
        compiler_params=pltpu.CompilerParams(
            dimension_semantics=("arbitrary",)),
    )(xf, W.T)

    return (gates.reshape(B, S, E), aux.reshape(()), gshard.reshape(()),
            imp.reshape(()))


def kernel(x, W):
    return _router(x, W)
